# baseline (speedup 1.0000x reference)
import dataclasses
import functools

import jax
import jax.numpy as jnp
from jax import lax
from jax.experimental import pallas as pl
from jax.experimental.pallas import tpu as pltpu
from jax.experimental.pallas import tpu_sc as plsc

N_NODES = 10000
E_EDGES = 160000
CH = 128
NCHUNKS = E_EDGES // CH
NUM_SC = 2
NUM_SUB = 16
NTILES = NUM_SC * NUM_SUB
ROWS_PER_SUB = N_NODES // NUM_SUB


def _matmul_body(x_ref, w_ref, o_ref):
    o_ref[...] = jnp.dot(x_ref[...], w_ref[...],
                         preferred_element_type=jnp.float32)


def _tc_matmul(x, w, bm):
    m, k = x.shape
    _, n = w.shape
    return pl.pallas_call(
        _matmul_body,
        grid=(m // bm,),
        in_specs=[pl.BlockSpec((bm, k), lambda i: (i, 0)),
                  pl.BlockSpec((k, n), lambda i: (0, 0))],
        out_specs=pl.BlockSpec((bm, n), lambda i: (i, 0)),
        out_shape=jax.ShapeDtypeStruct((m, n), jnp.float32),
    )(x, w)


def _fused_body(p0_ref, p1_ref, w_ref, o_ref):
    r = jnp.maximum(p0_ref[0] + p1_ref[0], 0.0)
    o_ref[...] = jnp.dot(r, w_ref[...], preferred_element_type=jnp.float32)


def _tc_add_relu_matmul(p, w, bm):
    _, m, k = p.shape
    _, n = w.shape
    return pl.pallas_call(
        _fused_body,
        grid=(m // bm,),
        in_specs=[pl.BlockSpec((1, bm, k), lambda i: (0, i, 0)),
                  pl.BlockSpec((1, bm, k), lambda i: (1, i, 0)),
                  pl.BlockSpec((k, n), lambda i: (0, 0))],
        out_specs=pl.BlockSpec((bm, n), lambda i: (i, 0)),
        out_shape=jax.ShapeDtypeStruct((m, n), jnp.float32),
    )(p, p, w)


def _add_body(a_ref, b_ref, o_ref):
    o_ref[...] = (a_ref[0] + b_ref[0]).T


def _tc_add_t(q):
    _, m, n = q.shape
    return pl.pallas_call(
        _add_body,
        grid=(1,),
        in_specs=[pl.BlockSpec((1, m, n), lambda i: (0, 0, 0)),
                  pl.BlockSpec((1, m, n), lambda i: (1, 0, 0))],
        out_specs=pl.BlockSpec((n, m), lambda i: (0, 0)),
        out_shape=jax.ShapeDtypeStruct((n, m), jnp.float32),
    )(q, q)


def _make_spmm(d):
    nvec = d // 16
    rps = 624
    tail = N_NODES - rps * NUM_SUB
    nfull = rps // CH
    rem = rps - nfull * CH
    nsteady = NCHUNKS // NTILES
    nleft = NCHUNKS - nsteady * NTILES
    RS = 3 if d >= 128 else 4
    ES = 4
    UN = 4 if d >= 128 else 8
    mesh = plsc.VectorSubcoreMesh(core_axis_name="c", subcore_axis_name="s")
    cp = pltpu.CompilerParams(needs_layout_passes=False,
                              use_tc_tiling_on_sc=False)

    @functools.partial(
        pl.kernel,
        compiler_params=cp,
        out_type=jax.ShapeDtypeStruct((NUM_SC, N_NODES, d), jnp.float32),
        mesh=mesh,
        scratch_types=(
            [pltpu.VMEM((2, CH), jnp.int32) for _ in range(ES)]
            + [pltpu.VMEM((1, CH), jnp.float32) for _ in range(ES)]
            + [pltpu.VMEM((CH, d), jnp.float32) for _ in range(RS)]
            + [pltpu.VMEM_SHARED((N_NODES, d), jnp.float32)]
            + [pltpu.SemaphoreType.DMA for _ in range(ES + 2 * RS)]
        ),
    )
    def spmm(h_hbm, ei_hbm, val_hbm, out_hbm, *scr):
        ebuf = scr[0:ES]
        vbuf = scr[ES:2 * ES]
        rows = scr[2 * ES:2 * ES + RS]
        acc = scr[2 * ES + RS]
        sems = scr[2 * ES + RS + 1:]
        semi = sems[0:ES]
        semg = sems[ES:ES + RS]
        sems_ = sems[ES + RS:ES + 2 * RS]
        rows0 = rows[0]
        eb0 = ebuf[0]
        ro0 = rows[0]
        cid = lax.axis_index("c")
        sid = lax.axis_index("s")
        wid = sid * NUM_SC + cid

        zidx = jnp.zeros((16,), jnp.int32)

        class _Handles:
            def __init__(self, hs):
                self.hs = hs

            def wait(self):
                for h in self.hs:
                    h.wait()

        def idx_start(k):
            s = k % ES
            e0 = (wid + k * NTILES) * CH
            return _Handles([
                pltpu.async_copy(ei_hbm.at[0, pl.ds(e0, CH)], ebuf[s].at[0],
                                 semi[s]),
                pltpu.async_copy(ei_hbm.at[1, pl.ds(e0, CH)], ebuf[s].at[1],
                                 semi[s]),
                pltpu.async_copy(val_hbm.at[pl.ds(e0, CH)], vbuf[s].at[0],
                                 semi[s]),
            ])

        def gather_start(k):
            e, s = k % ES, k % RS
            return pltpu.async_copy(h_hbm.at[ebuf[e].at[1]], rows[s], semg[s])

        def scale(e, s):
            @plsc.parallel_loop(0, CH, unroll=UN)
            def _(i):
                bc = plsc.load_gather(
                    vbuf[e], [zidx, jnp.full((16,), i, jnp.int32)])
                for j in range(nvec):
                    sl = pl.ds(j * 16, 16)
                    rows[s][i, sl] = rows[s][i, sl] * bc

        def scatter_start(k):
            e, s = k % ES, k % RS
            return pltpu.async_copy(rows[s], acc.at[ebuf[e].at[0]], sems_[s],
                                    add=True)

        hidx = [None] * nsteady
        hgat = [None] * nsteady
        hsct = [None] * nsteady
        hidx[0] = idx_start(0)
        if nsteady > 1:
            hidx[1] = idx_start(1)

        zsrc = rows[RS - 1]
        zero = jnp.zeros((16,), jnp.float32)

        @pl.loop(0, CH)
        def _(i):
            for j in range(nvec):
                zsrc[i, pl.ds(j * 16, 16)] = zero

        base = sid * rps
        for k in range(nfull):
            pltpu.sync_copy(zsrc, acc.at[pl.ds(base + k * CH, CH)])
        if rem:
            pltpu.sync_copy(zsrc.at[pl.ds(0, rem)],
                            acc.at[pl.ds(base + nfull * CH, rem)])

        @pl.when(sid == NUM_SUB - 1)
        def _():
            pltpu.sync_copy(zsrc.at[pl.ds(0, tail)],
                            acc.at[pl.ds(rps * NUM_SUB, tail)])

        hidx[0].wait()
        hgat[0] = gather_start(0)
        plsc.subcore_barrier()

        for k in range(nsteady):
            if k + 1 < nsteady:
                if k - 2 >= 0:
                    hsct[k - 2].wait()
                hidx[k + 1].wait()
                hgat[k + 1] = gather_start(k + 1)
            hgat[k].wait()
            scale(k % ES, k % RS)
            hsct[k] = scatter_start(k)
            if k + 2 < nsteady:
                hidx[k + 2] = idx_start(k + 2)
        for k in range(max(0, nsteady - 3), nsteady):
            if hsct[k] is not None and k + 3 >= nsteady:
                hsct[k].wait()

        @pl.when(wid < nleft)
        def _():
            e0 = (wid + nsteady * NTILES) * CH
            pltpu.sync_copy(ei_hbm.at[0, pl.ds(e0, CH)], eb0.at[0])
            pltpu.sync_copy(ei_hbm.at[1, pl.ds(e0, CH)], eb0.at[1])
            pltpu.sync_copy(val_hbm.at[pl.ds(e0, CH)], vbuf[0].at[0])
            pltpu.sync_copy(h_hbm.at[eb0.at[1]], ro0)
            scale(0, 0)
            pltpu.sync_copy(ro0, acc.at[eb0.at[0]], add=True)

        plsc.subcore_barrier()
        pltpu.sync_copy(acc.at[pl.ds(base, rps)],
                        out_hbm.at[cid, pl.ds(base, rps)])

        @pl.when(sid == NUM_SUB - 1)
        def _():
            pltpu.sync_copy(acc.at[pl.ds(rps * NUM_SUB, tail)],
                            out_hbm.at[cid, pl.ds(rps * NUM_SUB, tail)])

    return spmm


_spmm128 = _make_spmm(128)
_spmm64 = _make_spmm(64)


def kernel(x, edge_index, adj_values, W1, W2):
    ei = edge_index.astype(jnp.int32)
    h1 = _tc_matmul(x, W1, bm=2000)
    p = _spmm128(h1, ei, adj_values)
    h2 = _tc_add_relu_matmul(p, W2, bm=2000)
    q = _spmm64(h2, ei, adj_values)
    return _tc_add_t(q).T


# speedup vs baseline: 1.0183x; 1.0183x over previous
import dataclasses
import functools

import jax
import jax.numpy as jnp
from jax import lax
from jax.experimental import pallas as pl
from jax.experimental.pallas import tpu as pltpu
from jax.experimental.pallas import tpu_sc as plsc

N_NODES = 10000
E_EDGES = 160000
CH = 128
NCHUNKS = E_EDGES // CH
NUM_SC = 2
NUM_SUB = 16
NTILES = NUM_SC * NUM_SUB
ROWS_PER_SUB = N_NODES // NUM_SUB


def _matmul_body(x_ref, w_ref, o_ref):
    o_ref[...] = jnp.dot(x_ref[...].astype(jnp.bfloat16),
                         w_ref[...].astype(jnp.bfloat16),
                         preferred_element_type=jnp.float32)


def _tc_matmul(x, w, bm):
    m, k = x.shape
    _, n = w.shape
    return pl.pallas_call(
        _matmul_body,
        grid=(m // bm,),
        in_specs=[pl.BlockSpec((bm, k), lambda i: (i, 0)),
                  pl.BlockSpec((k, n), lambda i: (0, 0))],
        out_specs=pl.BlockSpec((bm, n), lambda i: (i, 0)),
        out_shape=jax.ShapeDtypeStruct((m, n), jnp.float32),
    )(x, w)


def _fused_body(p0_ref, p1_ref, w_ref, o_ref):
    r = jnp.maximum(p0_ref[0] + p1_ref[0], 0.0)
    o_ref[...] = jnp.dot(r, w_ref[...], preferred_element_type=jnp.float32)


def _tc_add_relu_matmul(p, w, bm):
    _, m, k = p.shape
    _, n = w.shape
    return pl.pallas_call(
        _fused_body,
        grid=(m // bm,),
        in_specs=[pl.BlockSpec((1, bm, k), lambda i: (0, i, 0)),
                  pl.BlockSpec((1, bm, k), lambda i: (1, i, 0)),
                  pl.BlockSpec((k, n), lambda i: (0, 0))],
        out_specs=pl.BlockSpec((bm, n), lambda i: (i, 0)),
        out_shape=jax.ShapeDtypeStruct((m, n), jnp.float32),
    )(p, p, w)


def _add_body(a_ref, b_ref, o_ref):
    o_ref[...] = (a_ref[0] + b_ref[0]).T


def _tc_add_t(q):
    _, m, n = q.shape
    return pl.pallas_call(
        _add_body,
        grid=(1,),
        in_specs=[pl.BlockSpec((1, m, n), lambda i: (0, 0, 0)),
                  pl.BlockSpec((1, m, n), lambda i: (1, 0, 0))],
        out_specs=pl.BlockSpec((n, m), lambda i: (0, 0)),
        out_shape=jax.ShapeDtypeStruct((n, m), jnp.float32),
    )(q, q)


def _make_spmm(d):
    nvec = d // 16
    rps = 624
    tail = N_NODES - rps * NUM_SUB
    nfull = rps // CH
    rem = rps - nfull * CH
    nsteady = NCHUNKS // NTILES
    nleft = NCHUNKS - nsteady * NTILES
    RS = 3 if d >= 128 else 4
    ES = 4
    UN = 4
    mesh = plsc.VectorSubcoreMesh(core_axis_name="c", subcore_axis_name="s")
    cp = pltpu.CompilerParams(needs_layout_passes=False,
                              use_tc_tiling_on_sc=False)

    @functools.partial(
        pl.kernel,
        compiler_params=cp,
        out_type=jax.ShapeDtypeStruct((NUM_SC, N_NODES, d), jnp.float32),
        mesh=mesh,
        scratch_types=(
            [pltpu.VMEM((2, CH), jnp.int32) for _ in range(ES)]
            + [pltpu.VMEM((1, CH), jnp.float32) for _ in range(ES)]
            + [pltpu.VMEM((CH, d), jnp.float32) for _ in range(RS)]
            + [pltpu.VMEM_SHARED((N_NODES, d), jnp.float32)]
            + [pltpu.SemaphoreType.DMA for _ in range(ES + 2 * RS)]
        ),
    )
    def spmm(h_hbm, ei_hbm, val_hbm, out_hbm, *scr):
        ebuf = scr[0:ES]
        vbuf = scr[ES:2 * ES]
        rows = scr[2 * ES:2 * ES + RS]
        acc = scr[2 * ES + RS]
        sems = scr[2 * ES + RS + 1:]
        semi = sems[0:ES]
        semg = sems[ES:ES + RS]
        sems_ = sems[ES + RS:ES + 2 * RS]
        rows0 = rows[0]
        eb0 = ebuf[0]
        ro0 = rows[0]
        cid = lax.axis_index("c")
        sid = lax.axis_index("s")
        wid = sid * NUM_SC + cid

        zidx = jnp.zeros((16,), jnp.int32)

        class _Handles:
            def __init__(self, hs):
                self.hs = hs

            def wait(self):
                for h in self.hs:
                    h.wait()

        def idx_start(k):
            s = k % ES
            e0 = (wid + k * NTILES) * CH
            return _Handles([
                pltpu.async_copy(ei_hbm.at[0, pl.ds(e0, CH)], ebuf[s].at[0],
                                 semi[s]),
                pltpu.async_copy(ei_hbm.at[1, pl.ds(e0, CH)], ebuf[s].at[1],
                                 semi[s]),
                pltpu.async_copy(val_hbm.at[pl.ds(e0, CH)], vbuf[s].at[0],
                                 semi[s]),
            ])

        def gather_start(k):
            e, s = k % ES, k % RS
            return pltpu.async_copy(h_hbm.at[ebuf[e].at[1]], rows[s], semg[s])

        def scale(e, s):
            @plsc.parallel_loop(0, CH, unroll=UN)
            def _(i):
                bc = plsc.load_gather(
                    vbuf[e], [zidx, jnp.full((16,), i, jnp.int32)])
                for j in range(nvec):
                    sl = pl.ds(j * 16, 16)
                    rows[s][i, sl] = rows[s][i, sl] * bc

        def scatter_start(k):
            e, s = k % ES, k % RS
            return pltpu.async_copy(rows[s], acc.at[ebuf[e].at[0]], sems_[s],
                                    add=True)

        hidx = [None] * nsteady
        hgat = [None] * nsteady
        hsct = [None] * nsteady
        hidx[0] = idx_start(0)
        if nsteady > 1:
            hidx[1] = idx_start(1)

        zsrc = rows[RS - 1]
        zero = jnp.zeros((16,), jnp.float32)

        @pl.loop(0, CH)
        def _(i):
            for j in range(nvec):
                zsrc[i, pl.ds(j * 16, 16)] = zero

        base = sid * rps
        for k in range(nfull):
            pltpu.sync_copy(zsrc, acc.at[pl.ds(base + k * CH, CH)])
        if rem:
            pltpu.sync_copy(zsrc.at[pl.ds(0, rem)],
                            acc.at[pl.ds(base + nfull * CH, rem)])

        @pl.when(sid == NUM_SUB - 1)
        def _():
            pltpu.sync_copy(zsrc.at[pl.ds(0, tail)],
                            acc.at[pl.ds(rps * NUM_SUB, tail)])

        hidx[0].wait()
        hgat[0] = gather_start(0)
        plsc.subcore_barrier()

        for k in range(nsteady):
            if k + 1 < nsteady:
                if k - 2 >= 0:
                    hsct[k - 2].wait()
                hidx[k + 1].wait()
                hgat[k + 1] = gather_start(k + 1)
            hgat[k].wait()
            scale(k % ES, k % RS)
            hsct[k] = scatter_start(k)
            if k + 2 < nsteady:
                hidx[k + 2] = idx_start(k + 2)
        for k in range(max(0, nsteady - 3), nsteady):
            if hsct[k] is not None and k + 3 >= nsteady:
                hsct[k].wait()

        @pl.when(wid < nleft)
        def _():
            e0 = (wid + nsteady * NTILES) * CH
            pltpu.sync_copy(ei_hbm.at[0, pl.ds(e0, CH)], eb0.at[0])
            pltpu.sync_copy(ei_hbm.at[1, pl.ds(e0, CH)], eb0.at[1])
            pltpu.sync_copy(val_hbm.at[pl.ds(e0, CH)], vbuf[0].at[0])
            pltpu.sync_copy(h_hbm.at[eb0.at[1]], ro0)
            scale(0, 0)
            pltpu.sync_copy(ro0, acc.at[eb0.at[0]], add=True)

        plsc.subcore_barrier()
        pltpu.sync_copy(acc.at[pl.ds(base, rps)],
                        out_hbm.at[cid, pl.ds(base, rps)])

        @pl.when(sid == NUM_SUB - 1)
        def _():
            pltpu.sync_copy(acc.at[pl.ds(rps * NUM_SUB, tail)],
                            out_hbm.at[cid, pl.ds(rps * NUM_SUB, tail)])

    return spmm


_spmm128 = _make_spmm(128)
_spmm64 = _make_spmm(64)


def kernel(x, edge_index, adj_values, W1, W2):
    ei = edge_index.astype(jnp.int32)
    h1 = _tc_matmul(x, W1, bm=2000)
    p = _spmm128(h1, ei, adj_values)
    h2 = _tc_add_relu_matmul(p, W2, bm=2000)
    q = _spmm64(h2, ei, adj_values)
    return _tc_add_t(q).T


# speedup vs baseline: 1.0204x; 1.0020x over previous
import dataclasses
import functools

import jax
import jax.numpy as jnp
from jax import lax
from jax.experimental import pallas as pl
from jax.experimental.pallas import tpu as pltpu
from jax.experimental.pallas import tpu_sc as plsc

N_NODES = 10000
E_EDGES = 160000
CH = 128
NCHUNKS = E_EDGES // CH
NUM_SC = 2
NUM_SUB = 16
NTILES = NUM_SC * NUM_SUB
ROWS_PER_SUB = N_NODES // NUM_SUB


def _matmul_body(x_ref, w_ref, o_ref):
    o_ref[...] = jnp.dot(x_ref[...], w_ref[...],
                         preferred_element_type=jnp.float32)


def _tc_matmul(x, w, bm):
    m, k = x.shape
    _, n = w.shape
    return pl.pallas_call(
        _matmul_body,
        grid=(m // bm,),
        in_specs=[pl.BlockSpec((bm, k), lambda i: (i, 0)),
                  pl.BlockSpec((k, n), lambda i: (0, 0))],
        out_specs=pl.BlockSpec((bm, n), lambda i: (i, 0)),
        out_shape=jax.ShapeDtypeStruct((m, n), jnp.float32),
    )(x, w)


def _fused_body(p0_ref, p1_ref, w_ref, o_ref):
    r = jnp.maximum(p0_ref[0] + p1_ref[0], 0.0)
    o_ref[...] = jnp.dot(r, w_ref[...], preferred_element_type=jnp.float32)


def _tc_add_relu_matmul(p, w, bm):
    _, m, k = p.shape
    _, n = w.shape
    return pl.pallas_call(
        _fused_body,
        grid=(m // bm,),
        in_specs=[pl.BlockSpec((1, bm, k), lambda i: (0, i, 0)),
                  pl.BlockSpec((1, bm, k), lambda i: (1, i, 0)),
                  pl.BlockSpec((k, n), lambda i: (0, 0))],
        out_specs=pl.BlockSpec((bm, n), lambda i: (i, 0)),
        out_shape=jax.ShapeDtypeStruct((m, n), jnp.float32),
    )(p, p, w)


def _add_body(a_ref, b_ref, o_ref):
    o_ref[...] = (a_ref[0] + b_ref[0]).T


def _tc_add_t(q):
    _, m, n = q.shape
    return pl.pallas_call(
        _add_body,
        grid=(1,),
        in_specs=[pl.BlockSpec((1, m, n), lambda i: (0, 0, 0)),
                  pl.BlockSpec((1, m, n), lambda i: (1, 0, 0))],
        out_specs=pl.BlockSpec((n, m), lambda i: (0, 0)),
        out_shape=jax.ShapeDtypeStruct((n, m), jnp.float32),
    )(q, q)


def _make_spmm(d):
    nvec = d // 16
    rps = 624
    tail = N_NODES - rps * NUM_SUB
    nfull = rps // CH
    rem = rps - nfull * CH
    nsteady = NCHUNKS // NTILES
    nleft = NCHUNKS - nsteady * NTILES
    RS = 3 if d >= 128 else 4
    ES = 4
    UN = 4
    mesh = plsc.VectorSubcoreMesh(core_axis_name="c", subcore_axis_name="s")
    cp = pltpu.CompilerParams(needs_layout_passes=False,
                              use_tc_tiling_on_sc=False)

    @functools.partial(
        pl.kernel,
        compiler_params=cp,
        out_type=jax.ShapeDtypeStruct((NUM_SC, N_NODES, d), jnp.float32),
        mesh=mesh,
        scratch_types=(
            [pltpu.VMEM((2, CH), jnp.int32) for _ in range(ES)]
            + [pltpu.VMEM((1, CH), jnp.float32) for _ in range(ES)]
            + [pltpu.VMEM((CH, d), jnp.float32) for _ in range(RS)]
            + [pltpu.VMEM_SHARED((N_NODES, d), jnp.float32)]
            + [pltpu.SemaphoreType.DMA for _ in range(ES + 2 * RS)]
        ),
    )
    def spmm(h_hbm, ei_hbm, val_hbm, out_hbm, *scr):
        ebuf = scr[0:ES]
        vbuf = scr[ES:2 * ES]
        rows = scr[2 * ES:2 * ES + RS]
        acc = scr[2 * ES + RS]
        sems = scr[2 * ES + RS + 1:]
        semi = sems[0:ES]
        semg = sems[ES:ES + RS]
        sems_ = sems[ES + RS:ES + 2 * RS]
        rows0 = rows[0]
        eb0 = ebuf[0]
        ro0 = rows[0]
        cid = lax.axis_index("c")
        sid = lax.axis_index("s")
        wid = sid * NUM_SC + cid

        zidx = jnp.zeros((16,), jnp.int32)

        class _Handles:
            def __init__(self, hs):
                self.hs = hs

            def wait(self):
                for h in self.hs:
                    h.wait()

        def idx_start(k):
            s = k % ES
            e0 = (wid + k * NTILES) * CH
            return _Handles([
                pltpu.async_copy(ei_hbm.at[0, pl.ds(e0, CH)], ebuf[s].at[0],
                                 semi[s]),
                pltpu.async_copy(ei_hbm.at[1, pl.ds(e0, CH)], ebuf[s].at[1],
                                 semi[s]),
                pltpu.async_copy(val_hbm.at[pl.ds(e0, CH)], vbuf[s].at[0],
                                 semi[s]),
            ])

        def gather_start(k):
            e, s = k % ES, k % RS
            return pltpu.async_copy(h_hbm.at[ebuf[e].at[1]], rows[s], semg[s])

        def scale(e, s):
            @plsc.parallel_loop(0, CH, unroll=UN)
            def _(i):
                bc = plsc.load_gather(
                    vbuf[e], [zidx, jnp.full((16,), i, jnp.int32)])
                for j in range(nvec):
                    sl = pl.ds(j * 16, 16)
                    rows[s][i, sl] = rows[s][i, sl] * bc

        def scatter_start(k):
            e, s = k % ES, k % RS
            return pltpu.async_copy(rows[s], acc.at[ebuf[e].at[0]], sems_[s],
                                    add=True)

        hidx = [None] * nsteady
        hgat = [None] * nsteady
        hsct = [None] * nsteady
        hidx[0] = idx_start(0)
        if nsteady > 1:
            hidx[1] = idx_start(1)

        zsrc = rows[RS - 1]
        zero = jnp.zeros((16,), jnp.float32)

        @pl.loop(0, CH)
        def _(i):
            for j in range(nvec):
                zsrc[i, pl.ds(j * 16, 16)] = zero

        base = sid * rps
        for k in range(nfull):
            pltpu.sync_copy(zsrc, acc.at[pl.ds(base + k * CH, CH)])
        if rem:
            pltpu.sync_copy(zsrc.at[pl.ds(0, rem)],
                            acc.at[pl.ds(base + nfull * CH, rem)])

        @pl.when(sid == NUM_SUB - 1)
        def _():
            pltpu.sync_copy(zsrc.at[pl.ds(0, tail)],
                            acc.at[pl.ds(rps * NUM_SUB, tail)])

        hidx[0].wait()
        hgat[0] = gather_start(0)
        plsc.subcore_barrier()

        for k in range(nsteady):
            if k + 1 < nsteady:
                if k - 2 >= 0:
                    hsct[k - 2].wait()
                hidx[k + 1].wait()
                hgat[k + 1] = gather_start(k + 1)
            hgat[k].wait()
            scale(k % ES, k % RS)
            hsct[k] = scatter_start(k)
            if k + 2 < nsteady:
                hidx[k + 2] = idx_start(k + 2)
        for k in range(max(0, nsteady - 3), nsteady):
            if hsct[k] is not None and k + 3 >= nsteady:
                hsct[k].wait()

        @pl.when(wid < nleft)
        def _():
            e0 = (wid + nsteady * NTILES) * CH
            pltpu.sync_copy(ei_hbm.at[0, pl.ds(e0, CH)], eb0.at[0])
            pltpu.sync_copy(ei_hbm.at[1, pl.ds(e0, CH)], eb0.at[1])
            pltpu.sync_copy(val_hbm.at[pl.ds(e0, CH)], vbuf[0].at[0])
            pltpu.sync_copy(h_hbm.at[eb0.at[1]], ro0)
            scale(0, 0)
            pltpu.sync_copy(ro0, acc.at[eb0.at[0]], add=True)

        plsc.subcore_barrier()
        pltpu.sync_copy(acc.at[pl.ds(base, rps)],
                        out_hbm.at[cid, pl.ds(base, rps)])

        @pl.when(sid == NUM_SUB - 1)
        def _():
            pltpu.sync_copy(acc.at[pl.ds(rps * NUM_SUB, tail)],
                            out_hbm.at[cid, pl.ds(rps * NUM_SUB, tail)])

    return spmm


_spmm128 = _make_spmm(128)
_spmm64 = _make_spmm(64)


def kernel(x, edge_index, adj_values, W1, W2):
    ei = edge_index.astype(jnp.int32)
    h1 = _tc_matmul(x, W1, bm=2000)
    p = _spmm128(h1, ei, adj_values)
    h2 = _tc_add_relu_matmul(p, W2, bm=2000)
    q = _spmm64(h2, ei, adj_values)
    return _tc_add_t(q).T
